# baseline (speedup 1.0000x reference)
import functools

import jax
import jax.numpy as jnp
from jax.experimental import pallas as pl
from jax.experimental.pallas import tpu as pltpu

_DIM = 64
_N_EMBED = 1024
_ROWS = 16
_COLS = 1024
_TOKENS = _ROWS * _COLS
_BR = 1
_BLK = _BR * _COLS
_NUM_CHUNKS = _ROWS // _BR


def _vq_body(x_ref, e_ref, q_ref, ind_ref, diff_ref, ppl_ref):
    e = e_ref[...]
    e_sq = jnp.sum(e * e, axis=0, keepdims=True)
    iota = jax.lax.broadcasted_iota(jnp.int32, (_BLK, _N_EMBED), 1)

    cnt = jnp.zeros((_N_EMBED,), dtype=jnp.float32)
    dsum = jnp.float32(0.0)
    for c in range(_NUM_CHUNKS):
        x = x_ref[c * _BR:(c + 1) * _BR].reshape(_BLK, _DIM)
        neg2_scores = jax.lax.dot_general(
            x * (-2.0), e, (((1,), (0,)), ((), ())),
            preferred_element_type=jnp.float32)
        x_sq = jnp.sum(x * x, axis=1, keepdims=True)
        dist = (x_sq + neg2_scores) + e_sq

        ind = jnp.argmin(dist, axis=1).astype(jnp.int32)
        onehot = (iota == ind[:, None]).astype(jnp.float32)
        q = jax.lax.dot_general(
            onehot, e, (((1,), (1,)), ((), ())),
            preferred_element_type=jnp.float32)

        q_ref[c * _BR:(c + 1) * _BR] = q.reshape(_BR, _COLS, _DIM)
        ind_ref[c * _BLK:(c + 1) * _BLK] = ind

        ones = jnp.ones((1, _BLK), dtype=jnp.float32)
        cnt = cnt + jax.lax.dot_general(
            ones, onehot, (((1,), (0,)), ((), ())),
            preferred_element_type=jnp.float32)[0]
        dsum = dsum + jnp.sum((q - x) ** 2)

    diff_ref[...] = dsum / float(_TOKENS * _DIM)
    p = cnt / float(_TOKENS)
    ent = jnp.sum(p * jnp.log(jnp.clip(p, 1e-7, None)))
    ppl_ref[...] = jnp.exp(-ent)


@functools.partial(jax.jit, static_argnames=())
def kernel(input, embed):
    q, ind, diff, ppl = pl.pallas_call(
        _vq_body,
        out_specs=[
            pl.BlockSpec(memory_space=pltpu.MemorySpace.VMEM),
            pl.BlockSpec(memory_space=pltpu.MemorySpace.VMEM),
            pl.BlockSpec(memory_space=pltpu.MemorySpace.SMEM),
            pl.BlockSpec(memory_space=pltpu.MemorySpace.SMEM),
        ],
        out_shape=[
            jax.ShapeDtypeStruct((_ROWS, _COLS, _DIM), jnp.float32),
            jax.ShapeDtypeStruct((_TOKENS,), jnp.int32),
            jax.ShapeDtypeStruct((), jnp.float32),
            jax.ShapeDtypeStruct((), jnp.float32),
        ],
    )(input, embed)
    return q, diff, ind.reshape(_ROWS, _COLS), ppl
